# Initial kernel scaffold; baseline (speedup 1.0000x reference)
import jax
import jax.numpy as jnp
from jax.experimental import pallas as pl


def kernel(theta, edge_index, u0, W0, b0, W1, b1, W2, b2, W3, b3):
    raise NotImplementedError("write your pallas kernel here")


# speedup vs baseline: 88.0685x; 88.0685x over previous
import functools

import jax
import jax.numpy as jnp
import numpy as np
from jax import lax
from jax.experimental import pallas as pl
from jax.experimental.pallas import tpu as pltpu
from jax.experimental.pallas import tpu_sc as plsc

TWO_PI = float(2.0 * np.pi)
HID = 128

T = 4096
DX = TWO_PI / T
INV_DX = T / TWO_PI

NW = 32
RPC = 8
EPC = RPC * 128


def _table_body(w0, b0, w1, b1, w2, b2, w3, b3, y_ref, dy_ref, *, bt):
    pid = pl.program_id(0)
    i = lax.broadcasted_iota(jnp.int32, (bt, 1), 0).astype(jnp.float32)
    x0 = (i + pid * bt) * DX

    def fnn(x):
        h = jnp.maximum(x * w0[...] + b0[...], 0.0)
        h = jnp.maximum(
            jnp.dot(h, w1[...], preferred_element_type=jnp.float32,
                    precision=lax.Precision.HIGHEST) + b1[...], 0.0)
        h = jnp.maximum(
            jnp.dot(h, w2[...], preferred_element_type=jnp.float32,
                    precision=lax.Precision.HIGHEST) + b2[...], 0.0)
        return jnp.dot(h, w3[...], preferred_element_type=jnp.float32,
                       precision=lax.Precision.HIGHEST) + b3[...]

    y0 = fnn(x0)
    y1 = fnn(x0 + DX)
    y_ref[...] = y0
    dy_ref[...] = y1 - y0


def _build_table(W0, b0, W1, b1, W2, b2, W3, b3):
    bt = 2048
    grid = T // bt
    wspec = lambda shp: pl.BlockSpec(shp, lambda i: (0, 0))
    return pl.pallas_call(
        functools.partial(_table_body, bt=bt),
        grid=(grid,),
        in_specs=[wspec((1, HID)), wspec((1, HID)),
                  wspec((HID, HID)), wspec((1, HID)),
                  wspec((HID, HID)), wspec((1, HID)),
                  wspec((HID, 1)), wspec((1, 1))],
        out_specs=[pl.BlockSpec((bt, 1), lambda i: (i, 0)),
                   pl.BlockSpec((bt, 1), lambda i: (i, 0))],
        out_shape=[jax.ShapeDtypeStruct((T, 1), jnp.float32),
                   jax.ShapeDtypeStruct((T, 1), jnp.float32)],
    )(W0, b0.reshape(1, HID), W1, b1.reshape(1, HID),
      W2, b2.reshape(1, HID), W3, b3.reshape(1, 1))


def _make_edge_kernel(n, e):
    rows = e // 128
    chunks = rows // RPC
    ztile = ((n + 16 - 1) // 16 + 800 - 1) // 800 * 800
    npad = ztile * 16
    zparts = ztile // 800
    mesh = plsc.VectorSubcoreMesh(core_axis_name="c", subcore_axis_name="s")

    @functools.partial(
        pl.kernel,
        out_type=[jax.ShapeDtypeStruct((2, npad), jnp.float32),
                  jax.ShapeDtypeStruct((2, npad), jnp.float32)],
        mesh=mesh,
        scratch_types=[
            pltpu.VMEM((n,), jnp.float32),
            pltpu.VMEM((T,), jnp.float32),
            pltpu.VMEM((T,), jnp.float32),
            pltpu.VMEM((EPC,), jnp.int32),
            pltpu.VMEM((RPC, 128), jnp.int32),
            pltpu.VMEM((RPC, 128), jnp.float32),
            pltpu.VMEM((RPC, 128), jnp.float32),
            pltpu.VMEM((800,), jnp.float32),
            pltpu.VMEM_SHARED((npad,), jnp.float32),
            pltpu.VMEM_SHARED((npad,), jnp.float32),
        ],
        compiler_params=pltpu.CompilerParams(needs_layout_passes=False),
    )
    def edge_kernel(theta_hbm, edges_hbm, ytab_hbm, dytab_hbm,
                    sums_out, counts_out,
                    theta_v, ytab_v, dytab_v, src_v, dst_v, m_v, ones_v, z_v,
                    sums_sh, counts_sh):
        cid = lax.axis_index("c")
        sid = lax.axis_index("s")
        wid = sid * 2 + cid

        pltpu.sync_copy(theta_hbm, theta_v)
        pltpu.sync_copy(ytab_hbm, ytab_v)
        pltpu.sync_copy(dytab_hbm, dytab_v)

        def zfill(i, c):
            z_v[pl.ds(i * 16, 16)] = jnp.zeros((16,), jnp.float32)
            return c
        lax.fori_loop(0, 800 // 16, zfill, 0)

        def ofill(i, c):
            ones_v[i // 8, pl.ds((i % 8) * 16, 16)] = jnp.full(
                (16,), 1.0, jnp.float32)
            return c
        lax.fori_loop(0, RPC * 8, ofill, 0)

        def zinit(k, c):
            off = sid * ztile + k * 800
            pltpu.sync_copy(z_v, sums_sh.at[pl.ds(off, 800)])
            pltpu.sync_copy(z_v, counts_sh.at[pl.ds(off, 800)])
            return c
        lax.fori_loop(0, zparts, zinit, 0)
        plsc.subcore_barrier()

        def chunk_body(c, carry):
            base = c * EPC
            pltpu.sync_copy(edges_hbm.at[0, pl.ds(base, EPC)], src_v)

            def drow(j, cc):
                pltpu.sync_copy(edges_hbm.at[1, pl.ds(base + j * 128, 128)],
                                dst_v.at[j])
                return cc
            lax.fori_loop(0, RPC, drow, 0)

            def vec_body(v, cc):
                s_idx = src_v[pl.ds(v * 16, 16)]
                d_idx = dst_v[v // 8, pl.ds((v % 8) * 16, 16)]
                ts = plsc.load_gather(theta_v, [s_idx])
                td = plsc.load_gather(theta_v, [d_idx])
                diff = td - ts
                dth = diff + jnp.where(diff < 0.0, TWO_PI, 0.0)
                u = dth * INV_DX
                ui = jnp.clip(u.astype(jnp.int32), 0, T - 1)
                fr = u - ui.astype(jnp.float32)
                y0 = plsc.load_gather(ytab_v, [ui])
                dy = plsc.load_gather(dytab_v, [ui])
                m_v[v // 8, pl.ds((v % 8) * 16, 16)] = y0 + fr * dy
                return cc
            lax.fori_loop(0, RPC * 8, vec_body, 0)

            def srow(j, cc):
                pltpu.sync_copy(m_v.at[j], sums_sh.at[dst_v.at[j]], add=True)
                pltpu.sync_copy(ones_v.at[j], counts_sh.at[dst_v.at[j]],
                                add=True)
                return cc
            lax.fori_loop(0, RPC, srow, 0)
            return carry

        nmine = (chunks - wid + NW - 1) // NW
        lax.fori_loop(0, nmine,
                      lambda i, c: chunk_body(wid + i * NW, c), 0)

        plsc.subcore_barrier()
        off = sid * ztile
        pltpu.sync_copy(sums_sh.at[pl.ds(off, ztile)],
                        sums_out.at[cid, pl.ds(off, ztile)])
        pltpu.sync_copy(counts_sh.at[pl.ds(off, ztile)],
                        counts_out.at[cid, pl.ds(off, ztile)])

    return edge_kernel


def _fin_body(u0_ref, theta_ref, s0_ref, s1_ref, c0_ref, c1_ref,
              tq_ref, vc_ref, vs_ref):
    s = s0_ref[...] + s1_ref[...]
    c = c0_ref[...] + c1_ref[...]
    tq_ref[...] = s / jnp.maximum(c, 1.0)
    th = theta_ref[...]
    u0 = u0_ref[0, 0]
    vc_ref[...] = u0 * jnp.cos(th)
    vs_ref[...] = u0 * jnp.sin(th)


def _finalize(u0, theta_pad, s0, s1, c0, c1):
    rows = theta_pad.shape[0]
    shp = jax.ShapeDtypeStruct((rows, 128), jnp.float32)
    return pl.pallas_call(
        _fin_body,
        out_shape=[shp, shp, shp],
    )(u0.reshape(1, 1), theta_pad, s0, s1, c0, c1)


def kernel(theta, edge_index, u0, W0, b0, W1, b1, W2, b2, W3, b3):
    n = theta.shape[0]
    e = edge_index.shape[1]

    ytab, dytab = _build_table(W0, b0, W1, b1, W2, b2, W3, b3)

    edge_kernel = _make_edge_kernel(n, e)
    sums, counts = edge_kernel(theta.reshape(-1), edge_index,
                               ytab.reshape(-1), dytab.reshape(-1))

    npad = sums.shape[1]
    rows = npad // 128
    theta_pad = jnp.pad(theta.reshape(-1), (0, npad - n)).reshape(rows, 128)
    tq, vc, vs = _finalize(jnp.asarray(u0, jnp.float32), theta_pad,
                           sums[0].reshape(rows, 128),
                           sums[1].reshape(rows, 128),
                           counts[0].reshape(rows, 128),
                           counts[1].reshape(rows, 128))
    torque = tq.reshape(npad, 1)[:n]
    v = jnp.concatenate([vc.reshape(npad, 1)[:n], vs.reshape(npad, 1)[:n]],
                        axis=1)
    return torque, v


# speedup vs baseline: 157.6637x; 1.7902x over previous
import functools

import jax
import jax.numpy as jnp
import numpy as np
from jax import lax
from jax.experimental import pallas as pl
from jax.experimental.pallas import tpu as pltpu
from jax.experimental.pallas import tpu_sc as plsc

TWO_PI = float(2.0 * np.pi)
HID = 128

T = 4096
DX = TWO_PI / T
INV_DX = T / TWO_PI

NW = 32
RPC = 16
EPC = RPC * 128


def _table_body(w0, b0, w1, b1, w2, b2, w3, b3, y_ref, dy_ref, *, bt):
    pid = pl.program_id(0)
    i = lax.broadcasted_iota(jnp.int32, (bt, 1), 0).astype(jnp.float32)
    x0 = (i + pid * bt) * DX

    def fnn(x):
        h = jnp.maximum(x * w0[...] + b0[...], 0.0)
        h = jnp.maximum(
            jnp.dot(h, w1[...], preferred_element_type=jnp.float32,
                    precision=lax.Precision.HIGHEST) + b1[...], 0.0)
        h = jnp.maximum(
            jnp.dot(h, w2[...], preferred_element_type=jnp.float32,
                    precision=lax.Precision.HIGHEST) + b2[...], 0.0)
        return jnp.dot(h, w3[...], preferred_element_type=jnp.float32,
                       precision=lax.Precision.HIGHEST) + b3[...]

    y0 = fnn(x0)
    y1 = fnn(x0 + DX)
    y_ref[...] = y0
    dy_ref[...] = y1 - y0


def _build_table(W0, b0, W1, b1, W2, b2, W3, b3):
    bt = 2048
    grid = T // bt
    wspec = lambda shp: pl.BlockSpec(shp, lambda i: (0, 0))
    return pl.pallas_call(
        functools.partial(_table_body, bt=bt),
        grid=(grid,),
        in_specs=[wspec((1, HID)), wspec((1, HID)),
                  wspec((HID, HID)), wspec((1, HID)),
                  wspec((HID, HID)), wspec((1, HID)),
                  wspec((HID, 1)), wspec((1, 1))],
        out_specs=[pl.BlockSpec((bt, 1), lambda i: (i, 0)),
                   pl.BlockSpec((bt, 1), lambda i: (i, 0))],
        out_shape=[jax.ShapeDtypeStruct((T, 1), jnp.float32),
                   jax.ShapeDtypeStruct((T, 1), jnp.float32)],
    )(W0, b0.reshape(1, HID), W1, b1.reshape(1, HID),
      W2, b2.reshape(1, HID), W3, b3.reshape(1, 1))


def _make_edge_kernel(n, e):
    rows = e // 128
    chunks = rows // RPC
    ztile = ((n + 16 - 1) // 16 + 800 - 1) // 800 * 800
    npad = ztile * 16
    zparts = ztile // 800
    mesh = plsc.VectorSubcoreMesh(core_axis_name="c", subcore_axis_name="s")

    @functools.partial(
        pl.kernel,
        out_type=[jax.ShapeDtypeStruct((2, npad), jnp.float32),
                  jax.ShapeDtypeStruct((2, npad), jnp.float32)],
        mesh=mesh,
        scratch_types=[
            pltpu.VMEM((n,), jnp.float32),
            pltpu.VMEM((T,), jnp.float32),
            pltpu.VMEM((T,), jnp.float32),
            pltpu.VMEM((RPC, 128), jnp.int32),
            pltpu.VMEM((RPC, 128), jnp.int32),
            pltpu.VMEM((RPC, 128), jnp.float32),
            pltpu.VMEM((RPC, 128), jnp.float32),
            pltpu.VMEM((800,), jnp.float32),
            pltpu.VMEM_SHARED((npad,), jnp.float32),
            pltpu.VMEM_SHARED((npad,), jnp.float32),
        ],
        compiler_params=pltpu.CompilerParams(needs_layout_passes=False),
    )
    def edge_kernel(theta_hbm, edges_hbm, ytab_hbm, dytab_hbm,
                    sums_out, counts_out,
                    theta_v, ytab_v, dytab_v, src_v, dst_v, m_v, ones_v, z_v,
                    sums_sh, counts_sh):
        cid = lax.axis_index("c")
        sid = lax.axis_index("s")
        wid = sid * 2 + cid

        pltpu.sync_copy(theta_hbm, theta_v)
        pltpu.sync_copy(ytab_hbm, ytab_v)
        pltpu.sync_copy(dytab_hbm, dytab_v)

        def zfill(i, c):
            z_v[pl.ds(i * 16, 16)] = jnp.zeros((16,), jnp.float32)
            return c
        lax.fori_loop(0, 800 // 16, zfill, 0)

        def ofill(i, c):
            ones_v[i // 8, pl.ds((i % 8) * 16, 16)] = jnp.full(
                (16,), 1.0, jnp.float32)
            return c
        lax.fori_loop(0, RPC * 8, ofill, 0)

        def zinit(k, c):
            off = sid * ztile + k * 800
            pltpu.sync_copy(z_v, sums_sh.at[pl.ds(off, 800)])
            pltpu.sync_copy(z_v, counts_sh.at[pl.ds(off, 800)])
            return c
        lax.fori_loop(0, zparts, zinit, 0)
        plsc.subcore_barrier()

        def chunk_body(c, carry):
            row0 = c * RPC
            pltpu.sync_copy(edges_hbm.at[0, pl.ds(row0, RPC)], src_v)
            pltpu.sync_copy(edges_hbm.at[1, pl.ds(row0, RPC)], dst_v)

            def vec_body(v, cc):
                s_idx = src_v[v // 8, pl.ds((v % 8) * 16, 16)]
                d_idx = dst_v[v // 8, pl.ds((v % 8) * 16, 16)]
                ts = plsc.load_gather(theta_v, [s_idx])
                td = plsc.load_gather(theta_v, [d_idx])
                diff = td - ts
                dth = diff + jnp.where(diff < 0.0, TWO_PI, 0.0)
                u = dth * INV_DX
                ui = jnp.clip(u.astype(jnp.int32), 0, T - 1)
                fr = u - ui.astype(jnp.float32)
                y0 = plsc.load_gather(ytab_v, [ui])
                dy = plsc.load_gather(dytab_v, [ui])
                m_v[v // 8, pl.ds((v % 8) * 16, 16)] = y0 + fr * dy
                return cc
            lax.fori_loop(0, RPC * 8, vec_body, 0)

            def srow(j, cc):
                pltpu.sync_copy(m_v.at[j], sums_sh.at[dst_v.at[j]], add=True)
                pltpu.sync_copy(ones_v.at[j], counts_sh.at[dst_v.at[j]],
                                add=True)
                return cc
            lax.fori_loop(0, RPC, srow, 0)
            return carry

        nmine = (chunks - wid + NW - 1) // NW
        lax.fori_loop(0, nmine,
                      lambda i, c: chunk_body(wid + i * NW, c), 0)

        plsc.subcore_barrier()
        off = sid * ztile
        pltpu.sync_copy(sums_sh.at[pl.ds(off, ztile)],
                        sums_out.at[cid, pl.ds(off, ztile)])
        pltpu.sync_copy(counts_sh.at[pl.ds(off, ztile)],
                        counts_out.at[cid, pl.ds(off, ztile)])

    return edge_kernel


def _fin_body(u0_ref, theta_ref, s0_ref, s1_ref, c0_ref, c1_ref,
              tq_ref, vc_ref, vs_ref):
    s = s0_ref[...] + s1_ref[...]
    c = c0_ref[...] + c1_ref[...]
    tq_ref[...] = s / jnp.maximum(c, 1.0)
    th = theta_ref[...]
    u0 = u0_ref[0, 0]
    vc_ref[...] = u0 * jnp.cos(th)
    vs_ref[...] = u0 * jnp.sin(th)


def _finalize(u0, theta_pad, s0, s1, c0, c1):
    rows = theta_pad.shape[0]
    shp = jax.ShapeDtypeStruct((rows, 128), jnp.float32)
    return pl.pallas_call(
        _fin_body,
        out_shape=[shp, shp, shp],
    )(u0.reshape(1, 1), theta_pad, s0, s1, c0, c1)


def kernel(theta, edge_index, u0, W0, b0, W1, b1, W2, b2, W3, b3):
    n = theta.shape[0]
    e = edge_index.shape[1]

    ytab, dytab = _build_table(W0, b0, W1, b1, W2, b2, W3, b3)

    edge_kernel = _make_edge_kernel(n, e)
    sums, counts = edge_kernel(theta.reshape(-1),
                               edge_index.reshape(2, e // 128, 128),
                               ytab.reshape(-1), dytab.reshape(-1))

    npad = sums.shape[1]
    rows = npad // 128
    theta_pad = jnp.pad(theta.reshape(-1), (0, npad - n)).reshape(rows, 128)
    tq, vc, vs = _finalize(jnp.asarray(u0, jnp.float32), theta_pad,
                           sums[0].reshape(rows, 128),
                           sums[1].reshape(rows, 128),
                           counts[0].reshape(rows, 128),
                           counts[1].reshape(rows, 128))
    torque = tq.reshape(npad, 1)[:n]
    v = jnp.concatenate([vc.reshape(npad, 1)[:n], vs.reshape(npad, 1)[:n]],
                        axis=1)
    return torque, v


# speedup vs baseline: 165.8572x; 1.0520x over previous
import functools

import jax
import jax.numpy as jnp
import numpy as np
from jax import lax
from jax.experimental import pallas as pl
from jax.experimental.pallas import tpu as pltpu
from jax.experimental.pallas import tpu_sc as plsc

TWO_PI = float(2.0 * np.pi)
HID = 128

T = 4096
DX = TWO_PI / T
INV_DX = T / TWO_PI

NW = 32
RPC = 8
EPC = RPC * 128


def _table_body(w0, b0, w1, b1, w2, b2, w3, b3, y_ref, dy_ref, *, bt):
    pid = pl.program_id(0)
    i = lax.broadcasted_iota(jnp.int32, (bt, 1), 0).astype(jnp.float32)
    x0 = (i + pid * bt) * DX

    def fnn(x):
        h = jnp.maximum(x * w0[...] + b0[...], 0.0)
        h = jnp.maximum(
            jnp.dot(h, w1[...], preferred_element_type=jnp.float32,
                    precision=lax.Precision.HIGHEST) + b1[...], 0.0)
        h = jnp.maximum(
            jnp.dot(h, w2[...], preferred_element_type=jnp.float32,
                    precision=lax.Precision.HIGHEST) + b2[...], 0.0)
        return jnp.dot(h, w3[...], preferred_element_type=jnp.float32,
                       precision=lax.Precision.HIGHEST) + b3[...]

    y0 = fnn(x0)
    y1 = fnn(x0 + DX)
    y_ref[...] = y0
    dy_ref[...] = y1 - y0


def _build_table(W0, b0, W1, b1, W2, b2, W3, b3):
    bt = 2048
    grid = T // bt
    wspec = lambda shp: pl.BlockSpec(shp, lambda i: (0, 0))
    return pl.pallas_call(
        functools.partial(_table_body, bt=bt),
        grid=(grid,),
        in_specs=[wspec((1, HID)), wspec((1, HID)),
                  wspec((HID, HID)), wspec((1, HID)),
                  wspec((HID, HID)), wspec((1, HID)),
                  wspec((HID, 1)), wspec((1, 1))],
        out_specs=[pl.BlockSpec((bt, 1), lambda i: (i, 0)),
                   pl.BlockSpec((bt, 1), lambda i: (i, 0))],
        out_shape=[jax.ShapeDtypeStruct((T, 1), jnp.float32),
                   jax.ShapeDtypeStruct((T, 1), jnp.float32)],
    )(W0, b0.reshape(1, HID), W1, b1.reshape(1, HID),
      W2, b2.reshape(1, HID), W3, b3.reshape(1, 1))


def _make_edge_kernel(n, e):
    rows = e // 128
    rpc = RPC
    while rows % rpc:
        rpc -= 8
    chunks = rows // rpc
    ztile = ((n + 16 - 1) // 16 + 800 - 1) // 800 * 800
    npad = ztile * 16
    zparts = ztile // 800
    mesh = plsc.VectorSubcoreMesh(core_axis_name="c", subcore_axis_name="s")

    @functools.partial(
        pl.kernel,
        out_type=[jax.ShapeDtypeStruct((2, npad), jnp.float32),
                  jax.ShapeDtypeStruct((2, npad), jnp.float32)],
        mesh=mesh,
        scratch_types=[
            pltpu.VMEM((n,), jnp.float32),
            pltpu.VMEM((T,), jnp.float32),
            pltpu.VMEM((T,), jnp.float32),
            pltpu.VMEM((rpc, 128), jnp.int32),
            pltpu.VMEM((2, rpc, 128), jnp.int32),
            pltpu.VMEM((2, rpc, 128), jnp.float32),
            pltpu.VMEM((1, 128), jnp.float32),
            pltpu.VMEM((800,), jnp.float32),
            pltpu.VMEM_SHARED((npad,), jnp.float32),
            pltpu.VMEM_SHARED((npad,), jnp.float32),
            pltpu.SemaphoreType.DMA,
        ],
        compiler_params=pltpu.CompilerParams(needs_layout_passes=False),
    )
    def edge_kernel(theta_hbm, edges_hbm, ytab_hbm, dytab_hbm,
                    sums_out, counts_out,
                    theta_v, ytab_v, dytab_v, src_v, dst_v, m_v, ones_v, z_v,
                    sums_sh, counts_sh, scat_sem):
        cid = lax.axis_index("c")
        sid = lax.axis_index("s")
        wid = sid * 2 + cid

        pltpu.sync_copy(theta_hbm, theta_v)
        pltpu.sync_copy(ytab_hbm, ytab_v)
        pltpu.sync_copy(dytab_hbm, dytab_v)

        def zfill(i, c):
            z_v[pl.ds(i * 16, 16)] = jnp.zeros((16,), jnp.float32)
            return c
        lax.fori_loop(0, 800 // 16, zfill, 0)

        def ofill(i, c):
            ones_v[0, pl.ds(i * 16, 16)] = jnp.full((16,), 1.0, jnp.float32)
            return c
        lax.fori_loop(0, 8, ofill, 0)

        def zinit(k, c):
            off = sid * ztile + k * 800
            pltpu.sync_copy(z_v, sums_sh.at[pl.ds(off, 800)])
            pltpu.sync_copy(z_v, counts_sh.at[pl.ds(off, 800)])
            return c
        lax.fori_loop(0, zparts, zinit, 0)
        plsc.subcore_barrier()

        def drain(b):
            def dwait(j, cc):
                pltpu.make_async_copy(
                    m_v.at[b, j], sums_sh.at[dst_v.at[b, j]], scat_sem).wait()
                pltpu.make_async_copy(
                    ones_v.at[0], counts_sh.at[dst_v.at[b, j]],
                    scat_sem).wait()
                return cc
            lax.fori_loop(0, rpc, dwait, 0)

        def chunk_body(i, carry):
            c = wid + i * NW
            b = lax.rem(i, 2)
            row0 = c * rpc

            pltpu.sync_copy(edges_hbm.at[0, pl.ds(row0, rpc)], src_v)
            pltpu.sync_copy(edges_hbm.at[1, pl.ds(row0, rpc)], dst_v.at[b])

            def vec_body(v, cc):
                s_idx = src_v[v // 8, pl.ds((v % 8) * 16, 16)]
                d_idx = dst_v[b, v // 8, pl.ds((v % 8) * 16, 16)]
                ts = plsc.load_gather(theta_v, [s_idx])
                td = plsc.load_gather(theta_v, [d_idx])
                diff = td - ts
                dth = diff + jnp.where(diff < 0.0, TWO_PI, 0.0)
                u = dth * INV_DX
                ui = jnp.clip(u.astype(jnp.int32), 0, T - 1)
                fr = u - ui.astype(jnp.float32)
                y0 = plsc.load_gather(ytab_v, [ui])
                dy = plsc.load_gather(dytab_v, [ui])
                m_v[b, v // 8, pl.ds((v % 8) * 16, 16)] = y0 + fr * dy
                return cc
            lax.fori_loop(0, rpc * 8, vec_body, 0)

            def srow(j, cc):
                pltpu.async_copy(m_v.at[b, j], sums_sh.at[dst_v.at[b, j]],
                                 scat_sem, add=True)
                pltpu.async_copy(ones_v.at[0], counts_sh.at[dst_v.at[b, j]],
                                 scat_sem, add=True)
                return cc
            lax.fori_loop(0, rpc, srow, 0)
            drain(b)
            return carry

        nmine = (chunks - wid + NW - 1) // NW
        lax.fori_loop(0, nmine, chunk_body, 0)

        plsc.subcore_barrier()
        off = sid * ztile
        pltpu.sync_copy(sums_sh.at[pl.ds(off, ztile)],
                        sums_out.at[cid, pl.ds(off, ztile)])
        pltpu.sync_copy(counts_sh.at[pl.ds(off, ztile)],
                        counts_out.at[cid, pl.ds(off, ztile)])

    return edge_kernel


def _fin_body(u0_ref, theta_ref, s0_ref, s1_ref, c0_ref, c1_ref,
              tq_ref, vc_ref, vs_ref):
    s = s0_ref[...] + s1_ref[...]
    c = c0_ref[...] + c1_ref[...]
    tq_ref[...] = s / jnp.maximum(c, 1.0)
    th = theta_ref[...]
    u0 = u0_ref[0, 0]
    vc_ref[...] = u0 * jnp.cos(th)
    vs_ref[...] = u0 * jnp.sin(th)


def _finalize(u0, theta_pad, s0, s1, c0, c1):
    rows = theta_pad.shape[0]
    shp = jax.ShapeDtypeStruct((rows, 128), jnp.float32)
    return pl.pallas_call(
        _fin_body,
        out_shape=[shp, shp, shp],
    )(u0.reshape(1, 1), theta_pad, s0, s1, c0, c1)


def kernel(theta, edge_index, u0, W0, b0, W1, b1, W2, b2, W3, b3):
    n = theta.shape[0]
    e = edge_index.shape[1]

    ytab, dytab = _build_table(W0, b0, W1, b1, W2, b2, W3, b3)

    edge_kernel = _make_edge_kernel(n, e)
    sums, counts = edge_kernel(theta.reshape(-1),
                               edge_index.reshape(2, e // 128, 128),
                               ytab.reshape(-1), dytab.reshape(-1))

    npad = sums.shape[1]
    rows = npad // 128
    theta_pad = jnp.pad(theta.reshape(-1), (0, npad - n)).reshape(rows, 128)
    tq, vc, vs = _finalize(jnp.asarray(u0, jnp.float32), theta_pad,
                           sums[0].reshape(rows, 128),
                           sums[1].reshape(rows, 128),
                           counts[0].reshape(rows, 128),
                           counts[1].reshape(rows, 128))
    torque = tq.reshape(npad, 1)[:n]
    v = jnp.concatenate([vc.reshape(npad, 1)[:n], vs.reshape(npad, 1)[:n]],
                        axis=1)
    return torque, v


# speedup vs baseline: 167.5505x; 1.0102x over previous
import functools

import jax
import jax.numpy as jnp
import numpy as np
from jax import lax
from jax.experimental import pallas as pl
from jax.experimental.pallas import tpu as pltpu
from jax.experimental.pallas import tpu_sc as plsc

TWO_PI = float(2.0 * np.pi)
HID = 128

T = 32768
BT = 2048
TT = T + BT
DX = TWO_PI / T
INV_DX = T / TWO_PI

NW = 32
NS = 16
RPC = 8


def _table_body(w0, b0, w1, b1, w2, b2, w3, b3, y_ref):
    pid = pl.program_id(0)
    i = lax.broadcasted_iota(jnp.int32, (BT, 1), 0).astype(jnp.float32)
    x = (i + pid * BT) * DX
    h = jnp.maximum(x * w0[...] + b0[...], 0.0)
    h = jnp.maximum(
        jnp.dot(h, w1[...], preferred_element_type=jnp.float32) + b1[...],
        0.0)
    h = jnp.maximum(
        jnp.dot(h, w2[...], preferred_element_type=jnp.float32) + b2[...],
        0.0)
    y_ref[...] = (jnp.dot(h, w3[...], preferred_element_type=jnp.float32)
                  + b3[...])


def _build_table(W0, b0, W1, b1, W2, b2, W3, b3):
    wspec = lambda shp: pl.BlockSpec(shp, lambda i: (0, 0))
    return pl.pallas_call(
        _table_body,
        grid=(TT // BT,),
        in_specs=[wspec((1, HID)), wspec((1, HID)),
                  wspec((HID, HID)), wspec((1, HID)),
                  wspec((HID, HID)), wspec((1, HID)),
                  wspec((HID, 1)), wspec((1, 1))],
        out_specs=pl.BlockSpec((BT, 1), lambda i: (i, 0)),
        out_shape=jax.ShapeDtypeStruct((TT, 1), jnp.float32),
    )(W0, b0.reshape(1, HID), W1, b1.reshape(1, HID),
      W2, b2.reshape(1, HID), W3, b3.reshape(1, 1))


def _make_edge_kernel(n, e):
    rows = e // 128
    rpc = RPC
    while rows % rpc:
        rpc -= 8
    chunks = rows // rpc
    ztile = ((n + NS - 1) // NS + 800 - 1) // 800 * 800
    npad = ztile * NS
    zparts = ztile // 800
    tslice = ((n + NS - 1) // NS + 7) // 8 * 8
    ntheta = tslice * NS
    mesh = plsc.VectorSubcoreMesh(core_axis_name="c", subcore_axis_name="s")

    @functools.partial(
        pl.kernel,
        out_type=[jax.ShapeDtypeStruct((2, npad), jnp.float32),
                  jax.ShapeDtypeStruct((2, npad), jnp.float32)],
        mesh=mesh,
        scratch_types=[
            pltpu.VMEM((TT,), jnp.float32),
            pltpu.VMEM((2, rpc, 128), jnp.int32),
            pltpu.VMEM((2, rpc, 128), jnp.int32),
            pltpu.VMEM((2, rpc, 128), jnp.float32),
            pltpu.VMEM((rpc, 128), jnp.float32),
            pltpu.VMEM((rpc, 128), jnp.float32),
            pltpu.VMEM((1, 128), jnp.float32),
            pltpu.VMEM((800,), jnp.float32),
            pltpu.VMEM((tslice,), jnp.float32),
            pltpu.VMEM_SHARED((ntheta,), jnp.float32),
            pltpu.VMEM_SHARED((npad,), jnp.float32),
            pltpu.VMEM_SHARED((npad,), jnp.float32),
            pltpu.SemaphoreType.DMA,
            pltpu.SemaphoreType.DMA,
            pltpu.SemaphoreType.DMA,
        ],
        compiler_params=pltpu.CompilerParams(needs_layout_passes=False),
    )
    def edge_kernel(theta_hbm, edges_hbm, ytab_hbm,
                    sums_out, counts_out,
                    ytab_v, src_v, dst_v, m_v, ts_v, td_v, ones_v, z_v, tst_v,
                    theta_sh, sums_sh, counts_sh, scat_sem, in_sem, g_sem):
        cid = lax.axis_index("c")
        sid = lax.axis_index("s")
        wid = sid * 2 + cid

        pltpu.sync_copy(ytab_hbm, ytab_v)
        pltpu.sync_copy(theta_hbm.at[pl.ds(sid * tslice, tslice)], tst_v)
        pltpu.sync_copy(tst_v, theta_sh.at[pl.ds(sid * tslice, tslice)])

        def zfill(i, c):
            z_v[pl.ds(i * 16, 16)] = jnp.zeros((16,), jnp.float32)
            return c
        lax.fori_loop(0, 800 // 16, zfill, 0)

        def ofill(i, c):
            ones_v[0, pl.ds(i * 16, 16)] = jnp.full((16,), 1.0, jnp.float32)
            return c
        lax.fori_loop(0, 8, ofill, 0)

        def zinit(k, c):
            off = sid * ztile + k * 800
            pltpu.sync_copy(z_v, sums_sh.at[pl.ds(off, 800)])
            pltpu.sync_copy(z_v, counts_sh.at[pl.ds(off, 800)])
            return c
        lax.fori_loop(0, zparts, zinit, 0)
        plsc.subcore_barrier()

        nmine = (chunks - wid + NW - 1) // NW

        def issue_inputs(i, b):
            row0 = (wid + i * NW) * rpc
            pltpu.async_copy(edges_hbm.at[0, pl.ds(row0, rpc)], src_v.at[b],
                             in_sem)
            pltpu.async_copy(edges_hbm.at[1, pl.ds(row0, rpc)], dst_v.at[b],
                             in_sem)

        @pl.when(nmine > 0)
        def _():
            issue_inputs(0, 0)

        def chunk_body(i, carry):
            b = lax.rem(i, 2)

            pltpu.make_async_copy(
                edges_hbm.at[0, pl.ds((wid + i * NW) * rpc, rpc)],
                src_v.at[b], in_sem).wait()
            pltpu.make_async_copy(
                edges_hbm.at[1, pl.ds((wid + i * NW) * rpc, rpc)],
                dst_v.at[b], in_sem).wait()

            @pl.when(i + 1 < nmine)
            def _():
                issue_inputs(i + 1, 1 - b)

            def grow(j, cc):
                pltpu.async_copy(theta_sh.at[src_v.at[b, j]], ts_v.at[j],
                                 g_sem)
                pltpu.async_copy(theta_sh.at[dst_v.at[b, j]], td_v.at[j],
                                 g_sem)
                return cc
            lax.fori_loop(0, rpc, grow, 0)

            def gdrain(j, cc):
                pltpu.make_async_copy(theta_sh.at[src_v.at[b, j]],
                                      ts_v.at[j], g_sem).wait()
                pltpu.make_async_copy(theta_sh.at[dst_v.at[b, j]],
                                      td_v.at[j], g_sem).wait()
                return cc
            lax.fori_loop(0, rpc, gdrain, 0)

            def vec_body(v0, cc):
                for u_ in range(4):
                    v = v0 * 4 + u_
                    ts = ts_v[v // 8, pl.ds((v % 8) * 16, 16)]
                    td = td_v[v // 8, pl.ds((v % 8) * 16, 16)]
                    diff = td - ts
                    dth = diff + jnp.where(diff < 0.0, TWO_PI, 0.0)
                    u = dth * INV_DX
                    ui = jnp.clip(u.astype(jnp.int32), 0, T - 1)
                    fr = u - ui.astype(jnp.float32)
                    y0 = plsc.load_gather(ytab_v, [ui])
                    y1 = plsc.load_gather(ytab_v, [ui + 1])
                    m_v[b, v // 8, pl.ds((v % 8) * 16, 16)] = (
                        y0 + fr * (y1 - y0))
                return cc
            lax.fori_loop(0, rpc * 2, vec_body, 0)

            def srow(j, cc):
                pltpu.async_copy(m_v.at[b, j], sums_sh.at[dst_v.at[b, j]],
                                 scat_sem, add=True)
                pltpu.async_copy(ones_v.at[0], counts_sh.at[dst_v.at[b, j]],
                                 scat_sem, add=True)
                return cc
            lax.fori_loop(0, rpc, srow, 0)

            def sdrain(j, cc):
                pltpu.make_async_copy(
                    m_v.at[b, j], sums_sh.at[dst_v.at[b, j]],
                    scat_sem).wait()
                pltpu.make_async_copy(
                    ones_v.at[0], counts_sh.at[dst_v.at[b, j]],
                    scat_sem).wait()
                return cc
            lax.fori_loop(0, rpc, sdrain, 0)
            return carry

        lax.fori_loop(0, nmine, chunk_body, 0)

        plsc.subcore_barrier()
        off = sid * ztile
        pltpu.sync_copy(sums_sh.at[pl.ds(off, ztile)],
                        sums_out.at[cid, pl.ds(off, ztile)])
        pltpu.sync_copy(counts_sh.at[pl.ds(off, ztile)],
                        counts_out.at[cid, pl.ds(off, ztile)])

    return edge_kernel, ntheta


def _fin_body(u0_ref, theta_ref, s0_ref, s1_ref, c0_ref, c1_ref,
              tq_ref, vc_ref, vs_ref):
    s = s0_ref[...] + s1_ref[...]
    c = c0_ref[...] + c1_ref[...]
    tq_ref[...] = s / jnp.maximum(c, 1.0)
    th = theta_ref[...]
    u0 = u0_ref[0, 0]
    vc_ref[...] = u0 * jnp.cos(th)
    vs_ref[...] = u0 * jnp.sin(th)


def _finalize(u0, theta_pad, s0, s1, c0, c1):
    rows = theta_pad.shape[0]
    shp = jax.ShapeDtypeStruct((rows, 128), jnp.float32)
    return pl.pallas_call(
        _fin_body,
        out_shape=[shp, shp, shp],
    )(u0.reshape(1, 1), theta_pad, s0, s1, c0, c1)


def kernel(theta, edge_index, u0, W0, b0, W1, b1, W2, b2, W3, b3):
    n = theta.shape[0]
    e = edge_index.shape[1]

    ytab = _build_table(W0, b0, W1, b1, W2, b2, W3, b3)

    edge_kernel, ntheta = _make_edge_kernel(n, e)
    theta_flat = theta.reshape(-1)
    sums, counts = edge_kernel(jnp.pad(theta_flat, (0, ntheta - n)),
                               edge_index.reshape(2, e // 128, 128),
                               ytab.reshape(-1))

    npad = sums.shape[1]
    rows = npad // 128
    theta_pad = jnp.pad(theta_flat, (0, npad - n)).reshape(rows, 128)
    tq, vc, vs = _finalize(jnp.asarray(u0, jnp.float32), theta_pad,
                           sums[0].reshape(rows, 128),
                           sums[1].reshape(rows, 128),
                           counts[0].reshape(rows, 128),
                           counts[1].reshape(rows, 128))
    torque = tq.reshape(npad, 1)[:n]
    v = jnp.concatenate([vc.reshape(npad, 1)[:n], vs.reshape(npad, 1)[:n]],
                        axis=1)
    return torque, v
